# baseline (speedup 1.0000x reference)
import functools

import jax
import jax.numpy as jnp
from jax import lax
from jax.experimental import pallas as pl
from jax.experimental.pallas import tpu as pltpu
from jax.experimental.pallas import tpu_sc as plsc

N_ROWS = 16384
N_COLS = 2626
N_OUT = 39
N_MU = 13
N_CAT = 26
CAT_BASE = 2 * N_MU
CAT_W = 100

NC = 2
NS = 16
L = 16
NW = NC * NS
ROWS_PER_W = N_ROWS // NW
BLK = L
NBLK = ROWS_PER_W // BLK


def _sc_body(in_hbm, out_hbm, buf0, buf1, out_v, sem0, sem1):
    wid = lax.axis_index("s") * NC + lax.axis_index("c")
    row0 = wid * ROWS_PER_W

    lane = lax.iota(jnp.int32, L)
    row_off = lane * N_COLS

    def dma_in(b, buf, sem):
        base = pl.multiple_of((row0 + b * BLK) * N_COLS, 8)
        return pltpu.make_async_copy(
            in_hbm.at[pl.ds(base, BLK * N_COLS)], buf, sem)

    def block_compute(b, buf):
        out_base = (b * BLK) * N_OUT + lane * N_OUT
        for j in range(N_MU):
            v = plsc.load_gather(buf, [row_off + (2 * j)])
            plsc.store_scatter(out_v, [out_base + j], v)

        def seg_body(k, carry_out):
            seg0 = row_off + (CAT_BASE + k * CAT_W)

            def col_body(c, carry):
                m, mi = carry
                v = plsc.load_gather(buf, [seg0 + c])
                upd = v > m
                m = jnp.where(upd, v, m)
                mi = jnp.where(upd, c, mi)
                return m, mi

            m0 = jnp.full((L,), -jnp.inf, jnp.float32)
            mi0 = jnp.zeros((L,), jnp.int32)
            _m, mi = lax.fori_loop(0, CAT_W, col_body, (m0, mi0))
            plsc.store_scatter(out_v, [out_base + N_MU + k],
                               mi.astype(jnp.float32))
            return carry_out

        lax.fori_loop(0, N_CAT, seg_body, 0)

    dma_in(0, buf0, sem0).start()
    dma_in(1, buf1, sem1).start()

    def body(i, _):
        b = 2 * i
        dma_in(b, buf0, sem0).wait()
        block_compute(b, buf0)

        @pl.when(b + 2 < NBLK)
        def _():
            dma_in(b + 2, buf0, sem0).start()

        dma_in(b + 1, buf1, sem1).wait()
        block_compute(b + 1, buf1)

        @pl.when(b + 3 < NBLK)
        def _():
            dma_in(b + 3, buf1, sem1).start()

        return 0

    lax.fori_loop(0, NBLK // 2, body, 0)

    out_off = pl.multiple_of(row0 * N_OUT, 8)
    pltpu.sync_copy(out_v, out_hbm.at[pl.ds(out_off, ROWS_PER_W * N_OUT)])


@jax.jit
def _sc_call(flat_in):
    mesh = plsc.VectorSubcoreMesh(core_axis_name="c", subcore_axis_name="s")
    return pl.kernel(
        _sc_body,
        out_type=jax.ShapeDtypeStruct((N_ROWS * N_OUT,), jnp.float32),
        mesh=mesh,
        compiler_params=pltpu.CompilerParams(needs_layout_passes=False),
        scratch_types=[
            pltpu.VMEM((BLK * N_COLS,), jnp.float32),
            pltpu.VMEM((BLK * N_COLS,), jnp.float32),
            pltpu.VMEM((ROWS_PER_W * N_OUT,), jnp.float32),
            pltpu.SemaphoreType.DMA,
            pltpu.SemaphoreType.DMA,
        ],
    )(flat_in)


def kernel(distr_params):
    flat = distr_params.reshape(N_ROWS * N_COLS)
    out = _sc_call(flat)
    return out.reshape(N_ROWS, N_OUT)


# speedup vs baseline: 1.4809x; 1.4809x over previous
import functools

import jax
import jax.numpy as jnp
from jax import lax
from jax.experimental import pallas as pl
from jax.experimental.pallas import tpu as pltpu
from jax.experimental.pallas import tpu_sc as plsc

N_ROWS = 16384
N_COLS = 2626
N_OUT = 39
N_MU = 13
N_CAT = 26
CAT_BASE = 2 * N_MU
CAT_W = 100
CHAINS = 5
CHAIN_W = CAT_W // CHAINS

NC = 2
NS = 16
L = 16
NW = NC * NS
ROWS_PER_W = N_ROWS // NW
BLK = L
NBLK = ROWS_PER_W // BLK


def _sc_body(in_hbm, out_hbm, buf0, buf1, out_v, sem0, sem1):
    wid = lax.axis_index("s") * NC + lax.axis_index("c")
    row0 = wid * ROWS_PER_W

    lane = lax.iota(jnp.int32, L)
    row_off = lane * N_COLS

    def dma_in(b, buf, sem):
        base = pl.multiple_of((row0 + b * BLK) * N_COLS, 8)
        return pltpu.make_async_copy(
            in_hbm.at[pl.ds(base, BLK * N_COLS)], buf, sem)

    def block_compute(b, buf):
        out_base = (b * BLK) * N_OUT + lane * N_OUT
        for j in range(N_MU):
            v = plsc.load_gather(buf, [row_off + (2 * j)])
            plsc.store_scatter(out_v, [out_base + j], v)

        def seg_body(k, carry_out):
            seg0 = row_off + (CAT_BASE + k * CAT_W)
            neg_inf = jnp.full((L,), -jnp.inf, jnp.float32)
            ms = [neg_inf] * CHAINS
            gs = [seg0 + (j * CHAIN_W) for j in range(CHAINS)]
            mis = list(gs)
            for c in range(CHAIN_W):
                for j in range(CHAINS):
                    v = plsc.load_gather(buf, [gs[j]])
                    upd = v > ms[j]
                    ms[j] = jnp.where(upd, v, ms[j])
                    mis[j] = jnp.where(upd, gs[j], mis[j])
                    if c + 1 < CHAIN_W:
                        gs[j] = gs[j] + 1
            m, mi = ms[CHAINS - 1], mis[CHAINS - 1]
            for j in range(CHAINS - 2, -1, -1):
                upd = ms[j] >= m
                m = jnp.where(upd, ms[j], m)
                mi = jnp.where(upd, mis[j], mi)
            col = (mi - seg0).astype(jnp.float32)
            plsc.store_scatter(out_v, [out_base + N_MU + k], col)
            return carry_out

        lax.fori_loop(0, N_CAT, seg_body, 0)

    dma_in(0, buf0, sem0).start()
    dma_in(1, buf1, sem1).start()

    def body(i, _):
        b = 2 * i
        dma_in(b, buf0, sem0).wait()
        block_compute(b, buf0)

        @pl.when(b + 2 < NBLK)
        def _():
            dma_in(b + 2, buf0, sem0).start()

        dma_in(b + 1, buf1, sem1).wait()
        block_compute(b + 1, buf1)

        @pl.when(b + 3 < NBLK)
        def _():
            dma_in(b + 3, buf1, sem1).start()

        return 0

    lax.fori_loop(0, NBLK // 2, body, 0)

    out_off = pl.multiple_of(row0 * N_OUT, 8)
    pltpu.sync_copy(out_v, out_hbm.at[pl.ds(out_off, ROWS_PER_W * N_OUT)])


@jax.jit
def _sc_call(flat_in):
    mesh = plsc.VectorSubcoreMesh(core_axis_name="c", subcore_axis_name="s")
    return pl.kernel(
        _sc_body,
        out_type=jax.ShapeDtypeStruct((N_ROWS * N_OUT,), jnp.float32),
        mesh=mesh,
        compiler_params=pltpu.CompilerParams(needs_layout_passes=False),
        scratch_types=[
            pltpu.VMEM((BLK * N_COLS,), jnp.float32),
            pltpu.VMEM((BLK * N_COLS,), jnp.float32),
            pltpu.VMEM((ROWS_PER_W * N_OUT,), jnp.float32),
            pltpu.SemaphoreType.DMA,
            pltpu.SemaphoreType.DMA,
        ],
    )(flat_in)


def kernel(distr_params):
    flat = distr_params.reshape(N_ROWS * N_COLS)
    out = _sc_call(flat)
    return out.reshape(N_ROWS, N_OUT)
